# Initial kernel scaffold; baseline (speedup 1.0000x reference)
import jax
import jax.numpy as jnp
from jax.experimental import pallas as pl


def kernel(atom_fea, nbr_fea, nbr_fea_idx, crystal_atom_idx, atom_types, emb_W, emb_b, fc_W, fc_b, bn1_g, bn1_b, bn2_g, bn2_b, ctf_W, ctf_b, fcs_W, fcs_b, out_W, out_b):
    raise NotImplementedError("write your pallas kernel here")


# speedup vs baseline: 2.1252x; 2.1252x over previous
import functools

import jax
import jax.numpy as jnp
from jax import lax
from jax.experimental import pallas as pl
from jax.experimental.pallas import tpu as pltpu
from jax.experimental.pallas import tpu_sc as plsc

_N = 10000
_M = 32
_A = 128
_NBR = 16
_NCONV = 3
_H = 192
_B = 100
_NI = 28
_CU = 29
_E = _N * _M
_NW = 32
_PW = _E // _NW
_CH = 400
_NCH = _PW // _CH

_T = 40
_GRID = _N // _T


def _softplus(x):
    return jnp.maximum(x, 0.0) + jnp.log(1.0 + jnp.exp(-jnp.abs(x)))


def _sigmoid(x):
    return 1.0 / (1.0 + jnp.exp(-x))


def _sc_gather(idx_flat, table):
    mesh = plsc.VectorSubcoreMesh(core_axis_name="c", subcore_axis_name="s")

    @functools.partial(
        pl.kernel,
        out_type=jax.ShapeDtypeStruct((_E, _A), jnp.float32),
        mesh=mesh,
        scratch_types=[
            pltpu.VMEM((_CH,), jnp.int32),
            pltpu.VMEM((_CH, _A), jnp.float32),
            pltpu.SemaphoreType.DMA,
        ],
    )
    def gk(idx_hbm, tab_hbm, out_hbm, idx_v, rows_v, sem):
        wid = lax.axis_index("s") * 2 + lax.axis_index("c")
        base = wid * _PW

        def body(c, carry):
            off = base + c * _CH
            pltpu.sync_copy(idx_hbm.at[pl.ds(off, _CH)], idx_v)
            pltpu.async_copy(tab_hbm.at[idx_v], rows_v, sem).wait()
            pltpu.sync_copy(rows_v, out_hbm.at[pl.ds(off, _CH)])
            return carry

        lax.fori_loop(0, _NCH, body, 0)

    return gk(idx_flat, table)


def _embed(atom_fea, emb_W, emb_b):
    tm = 2000

    def body(x_ref, w_ref, b_ref, o_ref):
        o_ref[...] = (
            jnp.dot(x_ref[...], w_ref[...], preferred_element_type=jnp.float32)
            + b_ref[...]
        )

    return pl.pallas_call(
        body,
        grid=(_N // tm,),
        in_specs=[
            pl.BlockSpec((tm, _A), lambda i: (i, 0)),
            pl.BlockSpec((_A, _A), lambda i: (0, 0)),
            pl.BlockSpec((1, _A), lambda i: (0, 0)),
        ],
        out_specs=pl.BlockSpec((tm, _A), lambda i: (i, 0)),
        out_shape=jax.ShapeDtypeStruct((_N, _A), jnp.float32),
    )(atom_fea, emb_W, emb_b.reshape(1, _A))


def _gated_tile(x_ref, g_ref, e_ref, w1_ref, w2_ref, w3_ref, b_ref):
    xw = jnp.dot(x_ref[...], w1_ref[...], preferred_element_type=jnp.float32)
    g2 = g_ref[...].reshape(_T * _M, _A)
    gw = jnp.dot(g2, w2_ref[...], preferred_element_type=jnp.float32)
    e2 = e_ref[...].reshape(_T * _M, _NBR)
    ew = jnp.dot(e2, w3_ref[...], preferred_element_type=jnp.float32)
    xrep = jnp.broadcast_to(xw[:, None, :], (_T, _M, 2 * _A)).reshape(_T * _M, 2 * _A)
    return gw + ew + b_ref[...] + xrep


def _conv_stats(x, gath3, nbr_fea, w, b):
    def body(x_ref, g_ref, e_ref, w1_ref, w2_ref, w3_ref, b_ref, o_ref):
        gated = _gated_tile(x_ref, g_ref, e_ref, w1_ref, w2_ref, w3_ref, b_ref)
        s = jnp.sum(gated, axis=0).reshape(1, 2 * _A)
        ss = jnp.sum(gated * gated, axis=0).reshape(1, 2 * _A)
        part = jnp.concatenate([s, ss, jnp.zeros((6, 2 * _A), jnp.float32)], axis=0)

        @pl.when(pl.program_id(0) == 0)
        def _():
            o_ref[...] = jnp.zeros_like(o_ref)

        o_ref[...] += part

    return pl.pallas_call(
        body,
        grid=(_GRID,),
        in_specs=[
            pl.BlockSpec((_T, _A), lambda i: (i, 0)),
            pl.BlockSpec((_T, _M, _A), lambda i: (i, 0, 0)),
            pl.BlockSpec((_T, _M, _NBR), lambda i: (i, 0, 0)),
            pl.BlockSpec((_A, 2 * _A), lambda i: (0, 0)),
            pl.BlockSpec((_A, 2 * _A), lambda i: (0, 0)),
            pl.BlockSpec((_NBR, 2 * _A), lambda i: (0, 0)),
            pl.BlockSpec((1, 2 * _A), lambda i: (0, 0)),
        ],
        out_specs=pl.BlockSpec((8, 2 * _A), lambda i: (0, 0)),
        out_shape=jax.ShapeDtypeStruct((8, 2 * _A), jnp.float32),
    )(x, gath3, nbr_fea, w[:_A], w[_A:2 * _A], w[2 * _A:], b.reshape(1, 2 * _A))


def _conv_pass2(x, gath3, nbr_fea, wf, bf):
    def body(x_ref, g_ref, e_ref, w1_ref, w2_ref, w3_ref, b_ref, o_ref, st_ref):
        gated = _gated_tile(x_ref, g_ref, e_ref, w1_ref, w2_ref, w3_ref, b_ref)
        filt = _sigmoid(gated[:, :_A])
        core = _softplus(gated[:, _A:])
        prod = (filt * core).reshape(_T, _M, _A)
        ns = jnp.sum(prod, axis=1)
        o_ref[...] = ns
        s = jnp.sum(ns, axis=0).reshape(1, _A)
        ss = jnp.sum(ns * ns, axis=0).reshape(1, _A)
        part = jnp.concatenate([s, ss, jnp.zeros((6, _A), jnp.float32)], axis=0)

        @pl.when(pl.program_id(0) == 0)
        def _():
            st_ref[...] = jnp.zeros_like(st_ref)

        st_ref[...] += part

    return pl.pallas_call(
        body,
        grid=(_GRID,),
        in_specs=[
            pl.BlockSpec((_T, _A), lambda i: (i, 0)),
            pl.BlockSpec((_T, _M, _A), lambda i: (i, 0, 0)),
            pl.BlockSpec((_T, _M, _NBR), lambda i: (i, 0, 0)),
            pl.BlockSpec((_A, 2 * _A), lambda i: (0, 0)),
            pl.BlockSpec((_A, 2 * _A), lambda i: (0, 0)),
            pl.BlockSpec((_NBR, 2 * _A), lambda i: (0, 0)),
            pl.BlockSpec((1, 2 * _A), lambda i: (0, 0)),
        ],
        out_specs=[
            pl.BlockSpec((_T, _A), lambda i: (i, 0)),
            pl.BlockSpec((8, _A), lambda i: (0, 0)),
        ],
        out_shape=[
            jax.ShapeDtypeStruct((_N, _A), jnp.float32),
            jax.ShapeDtypeStruct((8, _A), jnp.float32),
        ],
    )(x, gath3, nbr_fea, wf[:_A], wf[_A:2 * _A], wf[2 * _A:], bf)


def _bn_resid(x, ns, scale, shift):
    tm = 2000

    def body(x_ref, n_ref, sc_ref, sh_ref, o_ref):
        o_ref[...] = _softplus(x_ref[...] + n_ref[...] * sc_ref[...] + sh_ref[...])

    return pl.pallas_call(
        body,
        grid=(_N // tm,),
        in_specs=[
            pl.BlockSpec((tm, _A), lambda i: (i, 0)),
            pl.BlockSpec((tm, _A), lambda i: (i, 0)),
            pl.BlockSpec((1, _A), lambda i: (0, 0)),
            pl.BlockSpec((1, _A), lambda i: (0, 0)),
        ],
        out_specs=pl.BlockSpec((tm, _A), lambda i: (i, 0)),
        out_shape=jax.ShapeDtypeStruct((_N, _A), jnp.float32),
    )(x, ns, scale, shift)


def _pool_head(x3, t2, cw, cb, f0w, f0b, f1w, f1b, ow, ob):
    apc = _N // _B

    def body(x_ref, t_ref, cw_ref, cb_ref, f0w_ref, f0b_ref, f1w_ref, f1b_ref,
             ow_ref, ob_ref, o_ref):
        xv = x_ref[...]
        tv = t_ref[...]
        pools = []
        for eid in (_NI, _CU):
            mask = (tv == eid).astype(jnp.float32)
            cnt = jnp.sum(mask, axis=1)
            ssum = jnp.sum(xv * mask[:, :, None], axis=1)
            pooled = jnp.where(
                cnt[:, None] > 0, ssum / jnp.maximum(cnt, 1.0)[:, None], 0.0
            )
            pools.append(pooled)
        crys = _softplus(jnp.concatenate(pools, axis=1))
        crys = _softplus(
            jnp.dot(crys, cw_ref[...], preferred_element_type=jnp.float32)
            + cb_ref[...]
        )
        crys = _softplus(
            jnp.dot(crys, f0w_ref[...], preferred_element_type=jnp.float32)
            + f0b_ref[...]
        )
        crys = _softplus(
            jnp.dot(crys, f1w_ref[...], preferred_element_type=jnp.float32)
            + f1b_ref[...]
        )
        o_ref[...] = (
            jnp.dot(crys, ow_ref[...], preferred_element_type=jnp.float32)
            + ob_ref[...]
        )

    return pl.pallas_call(
        body,
        grid=(1,),
        in_specs=[
            pl.BlockSpec((_B, apc, _A), lambda i: (0, 0, 0)),
            pl.BlockSpec((_B, apc), lambda i: (0, 0)),
            pl.BlockSpec((2 * _A, _H), lambda i: (0, 0)),
            pl.BlockSpec((1, _H), lambda i: (0, 0)),
            pl.BlockSpec((_H, _H), lambda i: (0, 0)),
            pl.BlockSpec((1, _H), lambda i: (0, 0)),
            pl.BlockSpec((_H, _H), lambda i: (0, 0)),
            pl.BlockSpec((1, _H), lambda i: (0, 0)),
            pl.BlockSpec((_H, 1), lambda i: (0, 0)),
            pl.BlockSpec((1, 1), lambda i: (0, 0)),
        ],
        out_specs=pl.BlockSpec((_B, 1), lambda i: (0, 0)),
        out_shape=jax.ShapeDtypeStruct((_B, 1), jnp.float32),
    )(x3, t2, cw, cb, f0w, f0b, f1w, f1b, ow, ob)


def kernel(atom_fea, nbr_fea, nbr_fea_idx, crystal_atom_idx, atom_types,
           emb_W, emb_b, fc_W, fc_b, bn1_g, bn1_b, bn2_g, bn2_b,
           ctf_W, ctf_b, fcs_W, fcs_b, out_W, out_b):
    idx_flat = nbr_fea_idx.reshape(_E).astype(jnp.int32)
    x = _embed(atom_fea, emb_W, emb_b)
    n1 = jnp.float32(_E)
    n2 = jnp.float32(_N)
    for i in range(_NCONV):
        gath3 = _sc_gather(idx_flat, x).reshape(_N, _M, _A)
        w, b = fc_W[i], fc_b[i]
        st1 = _conv_stats(x, gath3, nbr_fea, w, b)
        mu1 = st1[0] / n1
        var1 = st1[1] / n1 - mu1 * mu1
        sc1 = bn1_g[i] / jnp.sqrt(var1 + 1e-5)
        wf = w * sc1[None, :]
        bf = (b - mu1) * sc1 + bn1_b[i]
        ns, st2 = _conv_pass2(x, gath3, nbr_fea, wf, bf.reshape(1, 2 * _A))
        mu2 = st2[0] / n2
        var2 = st2[1] / n2 - mu2 * mu2
        sc2 = bn2_g[i] / jnp.sqrt(var2 + 1e-5)
        sh2 = bn2_b[i] - mu2 * sc2
        x = _bn_resid(x, ns, sc2.reshape(1, _A), sh2.reshape(1, _A))
    x3 = x.reshape(_B, _N // _B, _A)
    t2 = atom_types.reshape(_B, _N // _B).astype(jnp.int32)
    return _pool_head(
        x3, t2, ctf_W, ctf_b.reshape(1, _H),
        fcs_W[0], fcs_b[0].reshape(1, _H), fcs_W[1], fcs_b[1].reshape(1, _H),
        out_W, out_b.reshape(1, 1),
    )


# speedup vs baseline: 2.9161x; 1.3722x over previous
import functools

import jax
import jax.numpy as jnp
from jax import lax
from jax.experimental import pallas as pl
from jax.experimental.pallas import tpu as pltpu
from jax.experimental.pallas import tpu_sc as plsc

_N = 10000
_M = 32
_A = 128
_NBR = 16
_NCONV = 3
_H = 192
_B = 100
_NI = 28
_CU = 29
_E = _N * _M
_NW = 32
_PW = _E // _NW
_CH = 400
_NCH = _PW // _CH

_T = 80
_GRID = _N // _T


def _softplus(x):
    return jnp.maximum(x, 0.0) + jnp.log(1.0 + jnp.exp(-jnp.abs(x)))


def _sigmoid(x):
    return 1.0 / (1.0 + jnp.exp(-x))


def _sc_gather(idx_flat, table):
    mesh = plsc.VectorSubcoreMesh(core_axis_name="c", subcore_axis_name="s")

    @functools.partial(
        pl.kernel,
        out_type=jax.ShapeDtypeStruct((_E, _A), jnp.float32),
        mesh=mesh,
        scratch_types=[
            pltpu.VMEM((_CH,), jnp.int32),
            pltpu.VMEM((_CH, _A), jnp.float32),
            pltpu.SemaphoreType.DMA,
        ],
    )
    def gk(idx_hbm, tab_hbm, out_hbm, idx_v, rows_v, sem):
        wid = lax.axis_index("s") * 2 + lax.axis_index("c")
        base = wid * _PW

        def body(c, carry):
            off = base + c * _CH
            pltpu.sync_copy(idx_hbm.at[pl.ds(off, _CH)], idx_v)
            pltpu.async_copy(tab_hbm.at[idx_v], rows_v, sem).wait()
            pltpu.sync_copy(rows_v, out_hbm.at[pl.ds(off, _CH)])
            return carry

        lax.fori_loop(0, _NCH, body, 0)

    return gk(idx_flat, table)


def _embed(atom_fea, emb_W, emb_b):
    tm = 2000

    def body(x_ref, w_ref, b_ref, o_ref):
        o_ref[...] = (
            jnp.dot(x_ref[...], w_ref[...], preferred_element_type=jnp.float32)
            + b_ref[...]
        )

    return pl.pallas_call(
        body,
        grid=(_N // tm,),
        in_specs=[
            pl.BlockSpec((tm, _A), lambda i: (i, 0)),
            pl.BlockSpec((_A, _A), lambda i: (0, 0)),
            pl.BlockSpec((1, _A), lambda i: (0, 0)),
        ],
        out_specs=pl.BlockSpec((tm, _A), lambda i: (i, 0)),
        out_shape=jax.ShapeDtypeStruct((_N, _A), jnp.float32),
    )(atom_fea, emb_W, emb_b.reshape(1, _A))


def _gated_tile(x_ref, g_ref, e_ref, w1_ref, w2_ref, w3_ref, b_ref):
    xw = jnp.dot(x_ref[...], w1_ref[...], preferred_element_type=jnp.float32)
    g2 = g_ref[...].reshape(_T * _M, _A).astype(jnp.bfloat16)
    gw = jnp.dot(g2, w2_ref[...], preferred_element_type=jnp.float32)
    e2 = e_ref[...].reshape(_T * _M, _NBR)
    ew = jnp.dot(e2, w3_ref[...], preferred_element_type=jnp.float32)
    xrep = jnp.broadcast_to(xw[:, None, :], (_T, _M, 2 * _A)).reshape(_T * _M, 2 * _A)
    return gw + ew + b_ref[...] + xrep


def _conv_stats(x, gath3, nbr_fea, w, b):
    def body(x_ref, g_ref, e_ref, w1_ref, w2_ref, w3_ref, b_ref, o_ref):
        gated = _gated_tile(x_ref, g_ref, e_ref, w1_ref, w2_ref, w3_ref, b_ref)
        s = jnp.sum(gated, axis=0).reshape(1, 2 * _A)
        ss = jnp.sum(gated * gated, axis=0).reshape(1, 2 * _A)
        part = jnp.concatenate([s, ss, jnp.zeros((6, 2 * _A), jnp.float32)], axis=0)

        @pl.when(pl.program_id(0) == 0)
        def _():
            o_ref[...] = jnp.zeros_like(o_ref)

        o_ref[...] += part

    return pl.pallas_call(
        body,
        grid=(_GRID,),
        in_specs=[
            pl.BlockSpec((_T, _A), lambda i: (i, 0)),
            pl.BlockSpec((_T, _M, _A), lambda i: (i, 0, 0)),
            pl.BlockSpec((_T, _M, _NBR), lambda i: (i, 0, 0)),
            pl.BlockSpec((_A, 2 * _A), lambda i: (0, 0)),
            pl.BlockSpec((_A, 2 * _A), lambda i: (0, 0)),
            pl.BlockSpec((_NBR, 2 * _A), lambda i: (0, 0)),
            pl.BlockSpec((1, 2 * _A), lambda i: (0, 0)),
        ],
        out_specs=pl.BlockSpec((8, 2 * _A), lambda i: (0, 0)),
        out_shape=jax.ShapeDtypeStruct((8, 2 * _A), jnp.float32),
    )(x, gath3, nbr_fea, w[:_A], w[_A:2 * _A], w[2 * _A:], b.reshape(1, 2 * _A))


def _conv_pass2(x, gath3, nbr_fea, wf, bf):
    def body(x_ref, g_ref, e_ref, w1_ref, w2_ref, w3_ref, b_ref, o_ref, st_ref):
        gated = _gated_tile(x_ref, g_ref, e_ref, w1_ref, w2_ref, w3_ref, b_ref)
        filt = _sigmoid(gated[:, :_A])
        core = _softplus(gated[:, _A:])
        prod = (filt * core).reshape(_T, _M, _A)
        ns = jnp.sum(prod, axis=1)
        o_ref[...] = ns
        s = jnp.sum(ns, axis=0).reshape(1, _A)
        ss = jnp.sum(ns * ns, axis=0).reshape(1, _A)
        part = jnp.concatenate([s, ss, jnp.zeros((6, _A), jnp.float32)], axis=0)

        @pl.when(pl.program_id(0) == 0)
        def _():
            st_ref[...] = jnp.zeros_like(st_ref)

        st_ref[...] += part

    return pl.pallas_call(
        body,
        grid=(_GRID,),
        in_specs=[
            pl.BlockSpec((_T, _A), lambda i: (i, 0)),
            pl.BlockSpec((_T, _M, _A), lambda i: (i, 0, 0)),
            pl.BlockSpec((_T, _M, _NBR), lambda i: (i, 0, 0)),
            pl.BlockSpec((_A, 2 * _A), lambda i: (0, 0)),
            pl.BlockSpec((_A, 2 * _A), lambda i: (0, 0)),
            pl.BlockSpec((_NBR, 2 * _A), lambda i: (0, 0)),
            pl.BlockSpec((1, 2 * _A), lambda i: (0, 0)),
        ],
        out_specs=[
            pl.BlockSpec((_T, _A), lambda i: (i, 0)),
            pl.BlockSpec((8, _A), lambda i: (0, 0)),
        ],
        out_shape=[
            jax.ShapeDtypeStruct((_N, _A), jnp.float32),
            jax.ShapeDtypeStruct((8, _A), jnp.float32),
        ],
    )(x, gath3, nbr_fea, wf[:_A], wf[_A:2 * _A], wf[2 * _A:], bf)


def _bn_resid(x, ns, scale, shift):
    tm = 2000

    def body(x_ref, n_ref, sc_ref, sh_ref, o_ref):
        o_ref[...] = _softplus(x_ref[...] + n_ref[...] * sc_ref[...] + sh_ref[...])

    return pl.pallas_call(
        body,
        grid=(_N // tm,),
        in_specs=[
            pl.BlockSpec((tm, _A), lambda i: (i, 0)),
            pl.BlockSpec((tm, _A), lambda i: (i, 0)),
            pl.BlockSpec((1, _A), lambda i: (0, 0)),
            pl.BlockSpec((1, _A), lambda i: (0, 0)),
        ],
        out_specs=pl.BlockSpec((tm, _A), lambda i: (i, 0)),
        out_shape=jax.ShapeDtypeStruct((_N, _A), jnp.float32),
    )(x, ns, scale, shift)


def _pool_head(x3, t2, cw, cb, f0w, f0b, f1w, f1b, ow, ob):
    apc = _N // _B

    def body(x_ref, t_ref, cw_ref, cb_ref, f0w_ref, f0b_ref, f1w_ref, f1b_ref,
             ow_ref, ob_ref, o_ref):
        xv = x_ref[...]
        tv = t_ref[...]
        pools = []
        for eid in (_NI, _CU):
            mask = (tv == eid).astype(jnp.float32)
            cnt = jnp.sum(mask, axis=1)
            ssum = jnp.sum(xv * mask[:, :, None], axis=1)
            pooled = jnp.where(
                cnt[:, None] > 0, ssum / jnp.maximum(cnt, 1.0)[:, None], 0.0
            )
            pools.append(pooled)
        crys = _softplus(jnp.concatenate(pools, axis=1))
        crys = _softplus(
            jnp.dot(crys, cw_ref[...], preferred_element_type=jnp.float32)
            + cb_ref[...]
        )
        crys = _softplus(
            jnp.dot(crys, f0w_ref[...], preferred_element_type=jnp.float32)
            + f0b_ref[...]
        )
        crys = _softplus(
            jnp.dot(crys, f1w_ref[...], preferred_element_type=jnp.float32)
            + f1b_ref[...]
        )
        o_ref[...] = (
            jnp.dot(crys, ow_ref[...], preferred_element_type=jnp.float32)
            + ob_ref[...]
        )

    return pl.pallas_call(
        body,
        grid=(1,),
        in_specs=[
            pl.BlockSpec((_B, apc, _A), lambda i: (0, 0, 0)),
            pl.BlockSpec((_B, apc), lambda i: (0, 0)),
            pl.BlockSpec((2 * _A, _H), lambda i: (0, 0)),
            pl.BlockSpec((1, _H), lambda i: (0, 0)),
            pl.BlockSpec((_H, _H), lambda i: (0, 0)),
            pl.BlockSpec((1, _H), lambda i: (0, 0)),
            pl.BlockSpec((_H, _H), lambda i: (0, 0)),
            pl.BlockSpec((1, _H), lambda i: (0, 0)),
            pl.BlockSpec((_H, 1), lambda i: (0, 0)),
            pl.BlockSpec((1, 1), lambda i: (0, 0)),
        ],
        out_specs=pl.BlockSpec((_B, 1), lambda i: (0, 0)),
        out_shape=jax.ShapeDtypeStruct((_B, 1), jnp.float32),
    )(x3, t2, cw, cb, f0w, f0b, f1w, f1b, ow, ob)


def kernel(atom_fea, nbr_fea, nbr_fea_idx, crystal_atom_idx, atom_types,
           emb_W, emb_b, fc_W, fc_b, bn1_g, bn1_b, bn2_g, bn2_b,
           ctf_W, ctf_b, fcs_W, fcs_b, out_W, out_b):
    idx_flat = nbr_fea_idx.reshape(_E).astype(jnp.int32)
    nbr_bf = nbr_fea.astype(jnp.bfloat16)
    x = _embed(atom_fea, emb_W, emb_b)
    n1 = jnp.float32(_E)
    n2 = jnp.float32(_N)
    for i in range(_NCONV):
        x_bf = x.astype(jnp.bfloat16)
        gath3 = _sc_gather(idx_flat, x).reshape(_N, _M, _A)
        w, b = fc_W[i], fc_b[i]
        w_bf = w.astype(jnp.bfloat16)
        st1 = _conv_stats(x_bf, gath3, nbr_bf, w_bf, b)
        mu1 = st1[0] / n1
        var1 = st1[1] / n1 - mu1 * mu1
        sc1 = bn1_g[i] / jnp.sqrt(var1 + 1e-5)
        wf = (w * sc1[None, :]).astype(jnp.bfloat16)
        bf = (b - mu1) * sc1 + bn1_b[i]
        ns, st2 = _conv_pass2(x_bf, gath3, nbr_bf, wf, bf.reshape(1, 2 * _A))
        mu2 = st2[0] / n2
        var2 = st2[1] / n2 - mu2 * mu2
        sc2 = bn2_g[i] / jnp.sqrt(var2 + 1e-5)
        sh2 = bn2_b[i] - mu2 * sc2
        x = _bn_resid(x, ns, sc2.reshape(1, _A), sh2.reshape(1, _A))
    x3 = x.reshape(_B, _N // _B, _A)
    t2 = atom_types.reshape(_B, _N // _B).astype(jnp.int32)
    return _pool_head(
        x3, t2, ctf_W, ctf_b.reshape(1, _H),
        fcs_W[0], fcs_b[0].reshape(1, _H), fcs_W[1], fcs_b[1].reshape(1, _H),
        out_W, out_b.reshape(1, 1),
    )
